# Initial kernel scaffold; baseline (speedup 1.0000x reference)
import jax
import jax.numpy as jnp
from jax.experimental import pallas as pl


def kernel(x, edge_index, edge_attr, W1a, b1a, W1b, b1b, W1c, b1c, W2a, b2a, W2b, b2b, Wh1, bh1, Wh2, bh2):
    raise NotImplementedError("write your pallas kernel here")


# speedup vs baseline: 58.3914x; 58.3914x over previous
import functools

import jax
import jax.numpy as jnp
from jax import lax
from jax.experimental import pallas as pl
from jax.experimental.pallas import tpu as pltpu
from jax.experimental.pallas import tpu_sc as plsc

_N = 100000
_E = 6400000
_GES = 8

_NW = 32
_EW = _E // _NW
_C = 8000
_NCHUNK = _EW // _C
_NPAD = 100096
_SLICE = _NPAD // 16

_BN = 10000
_GRID = _N // _BN


def _make_edge_agg():
    mesh = plsc.VectorSubcoreMesh(core_axis_name="c", subcore_axis_name="s")

    @functools.partial(
        pl.kernel,
        out_type=jax.ShapeDtypeStruct((2 * _NPAD,), jnp.float32),
        mesh=mesh,
        scratch_types=[
            pltpu.VMEM((_C,), jnp.int32),
            pltpu.VMEM((_C,), jnp.int32),
            pltpu.VMEM((_C,), jnp.float32),
            pltpu.VMEM((_C,), jnp.float32),
            pltpu.VMEM_SHARED((_NPAD,), jnp.float32),
            pltpu.SemaphoreType.DMA,
        ],
    )
    def edge_agg(g_hbm, src_hbm, dst_hbm, attr_hbm, out_hbm,
                 src_v, dst_v, attr_v, msg_v, acc_sh, sem):
        c = lax.axis_index("c")
        s = lax.axis_index("s")
        wid = s * 2 + c

        @functools.partial(plsc.parallel_loop, 0, _SLICE // 16)
        def _zero(i):
            msg_v[pl.ds(i * 16, 16)] = jnp.zeros((16,), jnp.float32)

        pltpu.sync_copy(msg_v.at[pl.ds(0, _SLICE)],
                        acc_sh.at[pl.ds(s * _SLICE, _SLICE)])
        plsc.subcore_barrier()

        def _chunk(k, carry):
            base = pl.multiple_of(wid * _EW + k * _C, 8)
            pltpu.sync_copy(src_hbm.at[pl.ds(base, _C)], src_v)
            pltpu.sync_copy(attr_hbm.at[pl.ds(base, _C)], attr_v)
            pltpu.sync_copy(dst_hbm.at[pl.ds(base, _C)], dst_v)
            pltpu.async_copy(g_hbm.at[src_v], msg_v, sem).wait()

            @functools.partial(plsc.parallel_loop, 0, _C // 16, unroll=8)
            def _mul(i):
                msg_v[pl.ds(i * 16, 16)] = (
                    msg_v[pl.ds(i * 16, 16)] * attr_v[pl.ds(i * 16, 16)]
                )

            pltpu.sync_copy(msg_v, acc_sh.at[dst_v], add=True)
            return carry

        lax.fori_loop(0, _NCHUNK, _chunk, 0)
        plsc.subcore_barrier()

        off = pl.multiple_of(c * _NPAD + s * _SLICE, 8)
        pltpu.sync_copy(acc_sh.at[pl.ds(s * _SLICE, _SLICE)],
                        msg_v.at[pl.ds(0, _SLICE)])
        pltpu.sync_copy(msg_v.at[pl.ds(0, _SLICE)],
                        out_hbm.at[pl.ds(off, _SLICE)])

    return edge_agg


_edge_agg_impl = None


def _edge_agg(g, src, dst, attr):
    global _edge_agg_impl
    if _edge_agg_impl is None:
        _edge_agg_impl = _make_edge_agg()
    return _edge_agg_impl(g, src, dst, attr)


def _mlp1(x, W1a, b1a, W1b, b1b, W1c, b1c):
    h = jnp.maximum(jnp.dot(x, W1a, preferred_element_type=jnp.float32) + b1a, 0.0)
    h = jnp.maximum(jnp.dot(h, W1b, preferred_element_type=jnp.float32) + b1b, 0.0)
    z = jnp.dot(h, W1c, preferred_element_type=jnp.float32) + b1c
    return 1.0 / (1.0 + jnp.exp(-z))


def _node_gate_body(x_ref, W1a, b1a, W1b, b1b, W1c, b1c, g_ref):
    g_ref[...] = _mlp1(x_ref[...], W1a[...], b1a[...], W1b[...], b1b[...],
                       W1c[...], b1c[...])


def _node_update_body(x_ref, pt_ref, W2a, b2a, W2b, b2b,
                      W1a, b1a, W1b, b1b, W1c, b1c, xn_ref, g_ref):
    agg = pt_ref[:, 0:1] + pt_ref[:, 1:2]
    tmp = jnp.concatenate([x_ref[...], agg], axis=1)
    c1 = jnp.maximum(jnp.dot(tmp, W2a[...], preferred_element_type=jnp.float32) + b2a[...], 0.0)
    c2 = jnp.maximum(jnp.dot(c1, W2b[...], preferred_element_type=jnp.float32) + b2b[...], 0.0)
    xn = jnp.concatenate([x_ref[:, 0:1], c2], axis=1)
    xn_ref[...] = xn
    g_ref[...] = _mlp1(xn, W1a[...], b1a[...], W1b[...], b1b[...],
                       W1c[...], b1c[...])


def _node_final_body(x_ref, pt_ref, W2a, b2a, W2b, b2b,
                     Wh1, bh1, Wh2, bh2, out_ref):
    agg = pt_ref[:, 0:1] + pt_ref[:, 1:2]
    tmp = jnp.concatenate([x_ref[...], agg], axis=1)
    c1 = jnp.maximum(jnp.dot(tmp, W2a[...], preferred_element_type=jnp.float32) + b2a[...], 0.0)
    c2 = jnp.maximum(jnp.dot(c1, W2b[...], preferred_element_type=jnp.float32) + b2b[...], 0.0)
    h = jnp.maximum(jnp.dot(c2, Wh1[...], preferred_element_type=jnp.float32) + bh1[...], 0.0)
    z = jnp.dot(h, Wh2[...], preferred_element_type=jnp.float32) + bh2[...]
    out_ref[...] = 1.0 / (1.0 + jnp.exp(-z))


def _row_spec(cols):
    return pl.BlockSpec((_BN, cols), lambda i: (i, 0))


def _full_spec(shape):
    ndim = len(shape)
    return pl.BlockSpec(shape, lambda i: (0,) * ndim)


def _wspecs(*ws):
    return [_full_spec(w.shape) for w in ws]


def _node_gate(x, W1a, b1a, W1b, b1b, W1c, b1c):
    return pl.pallas_call(
        _node_gate_body,
        grid=(_GRID,),
        in_specs=[_row_spec(9)] + _wspecs(W1a, b1a, W1b, b1b, W1c, b1c),
        out_specs=_row_spec(1),
        out_shape=jax.ShapeDtypeStruct((_N, 1), jnp.float32),
    )(x, W1a, b1a, W1b, b1b, W1c, b1c)


def _node_update(x, partsT, W2a, b2a, W2b, b2b, W1a, b1a, W1b, b1b, W1c, b1c):
    return pl.pallas_call(
        _node_update_body,
        grid=(_GRID,),
        in_specs=[_row_spec(9), _row_spec(2)]
        + _wspecs(W2a, b2a, W2b, b2b, W1a, b1a, W1b, b1b, W1c, b1c),
        out_specs=[_row_spec(9), _row_spec(1)],
        out_shape=[
            jax.ShapeDtypeStruct((_N, 9), jnp.float32),
            jax.ShapeDtypeStruct((_N, 1), jnp.float32),
        ],
    )(x, partsT, W2a, b2a, W2b, b2b, W1a, b1a, W1b, b1b, W1c, b1c)


def _node_final(x, partsT, W2a, b2a, W2b, b2b, Wh1, bh1, Wh2, bh2):
    return pl.pallas_call(
        _node_final_body,
        grid=(_GRID,),
        in_specs=[_row_spec(9), _row_spec(2)]
        + _wspecs(W2a, b2a, W2b, b2b, Wh1, bh1, Wh2, bh2),
        out_specs=_row_spec(1),
        out_shape=jax.ShapeDtypeStruct((_N, 1), jnp.float32),
    )(x, partsT, W2a, b2a, W2b, b2b, Wh1, bh1, Wh2, bh2)


def kernel(x, edge_index, edge_attr, W1a, b1a, W1b, b1b, W1c, b1c,
           W2a, b2a, W2b, b2b, Wh1, bh1, Wh2, bh2):
    src = edge_index[0]
    dst = edge_index[1]
    attr = edge_attr.reshape(_E)

    b1a2 = b1a.reshape(1, -1)
    b1b2 = b1b.reshape(1, -1)
    b1c2 = b1c.reshape(1, -1)
    b2a2 = b2a.reshape(1, -1)
    b2b2 = b2b.reshape(1, -1)
    bh12 = bh1.reshape(1, -1)
    bh22 = bh2.reshape(1, -1)

    g = _node_gate(x, W1a, b1a2, W1b, b1b2, W1c, b1c2)

    xk = x
    for _ in range(2):
        parts = _edge_agg(g.reshape(_N), src, dst, attr)
        partsT = parts.reshape(2, _NPAD).T[:_N]
        xk, g = _node_update(xk, partsT, W2a, b2a2, W2b, b2b2,
                             W1a, b1a2, W1b, b1b2, W1c, b1c2)

    parts = _edge_agg(g.reshape(_N), src, dst, attr)
    partsT = parts.reshape(2, _NPAD).T[:_N]
    return _node_final(xk, partsT, W2a, b2a2, W2b, b2b2, Wh1, bh12, Wh2, bh22)


# speedup vs baseline: 91.1510x; 1.5610x over previous
import functools

import jax
import jax.numpy as jnp
from jax import lax
from jax.experimental import pallas as pl
from jax.experimental.pallas import tpu as pltpu
from jax.experimental.pallas import tpu_sc as plsc

_N = 100000
_E = 6400000
_GES = 8

_NW = 32
_EW = _E // _NW
_C = 8000
_NCHUNK = _EW // _C
_NPAD = 100096
_SLICE = _NPAD // 16

_BN = 10000
_GRID = _N // _BN


def _make_edge_agg():
    mesh = plsc.VectorSubcoreMesh(core_axis_name="c", subcore_axis_name="s")

    @functools.partial(
        pl.kernel,
        out_type=(
            jax.ShapeDtypeStruct((_NPAD,), jnp.float32),
            jax.ShapeDtypeStruct((_NPAD,), jnp.float32),
        ),
        mesh=mesh,
        scratch_types=[
            pltpu.VMEM((_C,), jnp.int32),
            pltpu.VMEM((_C,), jnp.int32),
            pltpu.VMEM((_C,), jnp.int32),
            pltpu.VMEM((_C,), jnp.int32),
            pltpu.VMEM((_C,), jnp.float32),
            pltpu.VMEM((_C,), jnp.float32),
            pltpu.VMEM((_C,), jnp.float32),
            pltpu.VMEM((_C,), jnp.float32),
            pltpu.VMEM_SHARED((_NPAD,), jnp.float32),
            pltpu.VMEM_SHARED((_NPAD,), jnp.float32),
            pltpu.SemaphoreType.DMA,
            pltpu.SemaphoreType.DMA,
            pltpu.SemaphoreType.DMA,
        ],
    )
    def edge_agg(g_hbm, src_hbm, dst_hbm, attr_hbm, out0_hbm, out1_hbm,
                 src0, src1, dst0, dst1, attr0, attr1, msg0, msg1,
                 acc_sh, g_sh, lin0, lin1, ga):
        c = lax.axis_index("c")
        s = lax.axis_index("s")
        wid = s * 2 + c
        ebase = wid * _EW
        SRC = [src0, src1]
        DST = [dst0, dst1]
        ATTR = [attr0, attr1]
        MSG = [msg0, msg1]
        LIN = [lin0, lin1]

        pltpu.sync_copy(g_hbm.at[pl.ds(s * _SLICE, _SLICE)],
                        msg0.at[pl.ds(0, _SLICE)])
        pltpu.sync_copy(msg0.at[pl.ds(0, _SLICE)],
                        g_sh.at[pl.ds(s * _SLICE, _SLICE)])

        @functools.partial(plsc.parallel_loop, 0, _SLICE // 16 + 1)
        def _zero(i):
            msg0[pl.ds(i * 16, 16)] = jnp.zeros((16,), jnp.float32)

        pltpu.sync_copy(msg0.at[pl.ds(0, _SLICE)],
                        acc_sh.at[pl.ds(s * _SLICE, _SLICE)])
        plsc.subcore_barrier()

        def _lin_start(k, b):
            base = pl.multiple_of(ebase + k * _C, 8)
            pltpu.async_copy(src_hbm.at[pl.ds(base, _C)], SRC[b], LIN[b])
            pltpu.async_copy(attr_hbm.at[pl.ds(base, _C)], ATTR[b], LIN[b])
            pltpu.async_copy(dst_hbm.at[pl.ds(base, _C)], DST[b], LIN[b])

        def _lin_wait(k, b):
            base = pl.multiple_of(ebase + k * _C, 8)
            pltpu.make_async_copy(src_hbm.at[pl.ds(base, _C)], SRC[b], LIN[b]).wait()
            pltpu.make_async_copy(attr_hbm.at[pl.ds(base, _C)], ATTR[b], LIN[b]).wait()
            pltpu.make_async_copy(dst_hbm.at[pl.ds(base, _C)], DST[b], LIN[b]).wait()

        def _substep(k, b, issue_next):
            _lin_wait(k, b)
            gather = pltpu.async_copy(g_sh.at[SRC[b]], MSG[b], ga)
            if issue_next:
                _lin_start(k + 1, 1 - b)
            gather.wait()

            @functools.partial(plsc.parallel_loop, 0, _C // 16, unroll=8)
            def _mul(i):
                MSG[b][pl.ds(i * 16, 16)] = (
                    MSG[b][pl.ds(i * 16, 16)] * ATTR[b][pl.ds(i * 16, 16)]
                )

            pltpu.sync_copy(MSG[b], acc_sh.at[DST[b]], add=True)

        _lin_start(0, 0)
        _substep(0, 0, issue_next=True)

        def _pair(j, carry):
            _substep(2 * j + 1, 1, issue_next=True)
            _substep(2 * j + 2, 0, issue_next=True)
            return carry

        lax.fori_loop(0, (_NCHUNK - 3) // 2, _pair, 0)

        _substep(_NCHUNK - 2, 1, issue_next=True)
        _substep(_NCHUNK - 1, 0, issue_next=False)
        plsc.subcore_barrier()

        def _drain(out_hbm):
            pltpu.sync_copy(acc_sh.at[pl.ds(s * _SLICE, _SLICE)],
                            msg0.at[pl.ds(0, _SLICE)])
            pltpu.sync_copy(msg0.at[pl.ds(0, _SLICE)],
                            out_hbm.at[pl.ds(s * _SLICE, _SLICE)])

        @pl.when(c == 0)
        def _():
            _drain(out0_hbm)

        @pl.when(c == 1)
        def _():
            _drain(out1_hbm)

    return edge_agg


_edge_agg_impl = None


def _edge_agg(g, src, dst, attr):
    global _edge_agg_impl
    if _edge_agg_impl is None:
        _edge_agg_impl = _make_edge_agg()
    return _edge_agg_impl(g, src, dst, attr)


def _mlp1(x, W1a, b1a, W1b, b1b, W1c, b1c):
    h = jnp.maximum(jnp.dot(x, W1a, preferred_element_type=jnp.float32) + b1a, 0.0)
    h = jnp.maximum(jnp.dot(h, W1b, preferred_element_type=jnp.float32) + b1b, 0.0)
    z = jnp.dot(h, W1c, preferred_element_type=jnp.float32) + b1c
    return 1.0 / (1.0 + jnp.exp(-z))


def _node_gate_body(x_ref, W1a, b1a, W1b, b1b, W1c, b1c, g_ref):
    g_ref[...] = _mlp1(x_ref[...], W1a[...], b1a[...], W1b[...], b1b[...],
                       W1c[...], b1c[...])


def _node_update_body(x_ref, p0_ref, p1_ref, W2a, b2a, W2b, b2b,
                      W1a, b1a, W1b, b1b, W1c, b1c, xn_ref, g_ref):
    agg = p0_ref[...] + p1_ref[...]
    tmp = jnp.concatenate([x_ref[...], agg], axis=1)
    c1 = jnp.maximum(jnp.dot(tmp, W2a[...], preferred_element_type=jnp.float32) + b2a[...], 0.0)
    c2 = jnp.maximum(jnp.dot(c1, W2b[...], preferred_element_type=jnp.float32) + b2b[...], 0.0)
    xn = jnp.concatenate([x_ref[:, 0:1], c2], axis=1)
    xn_ref[...] = xn
    g_ref[...] = _mlp1(xn, W1a[...], b1a[...], W1b[...], b1b[...],
                       W1c[...], b1c[...])


def _node_final_body(x_ref, p0_ref, p1_ref, W2a, b2a, W2b, b2b,
                     Wh1, bh1, Wh2, bh2, out_ref):
    agg = p0_ref[...] + p1_ref[...]
    tmp = jnp.concatenate([x_ref[...], agg], axis=1)
    c1 = jnp.maximum(jnp.dot(tmp, W2a[...], preferred_element_type=jnp.float32) + b2a[...], 0.0)
    c2 = jnp.maximum(jnp.dot(c1, W2b[...], preferred_element_type=jnp.float32) + b2b[...], 0.0)
    h = jnp.maximum(jnp.dot(c2, Wh1[...], preferred_element_type=jnp.float32) + bh1[...], 0.0)
    z = jnp.dot(h, Wh2[...], preferred_element_type=jnp.float32) + bh2[...]
    out_ref[...] = 1.0 / (1.0 + jnp.exp(-z))


def _row_spec(cols):
    return pl.BlockSpec((_BN, cols), lambda i: (i, 0))


def _full_spec(shape):
    ndim = len(shape)
    return pl.BlockSpec(shape, lambda i: (0,) * ndim)


def _wspecs(*ws):
    return [_full_spec(w.shape) for w in ws]


def _node_gate(x, W1a, b1a, W1b, b1b, W1c, b1c):
    return pl.pallas_call(
        _node_gate_body,
        grid=(_GRID,),
        in_specs=[_row_spec(9)] + _wspecs(W1a, b1a, W1b, b1b, W1c, b1c),
        out_specs=_row_spec(1),
        out_shape=jax.ShapeDtypeStruct((_NPAD, 1), jnp.float32),
    )(x, W1a, b1a, W1b, b1b, W1c, b1c)


def _node_update(x, p0, p1, W2a, b2a, W2b, b2b, W1a, b1a, W1b, b1b, W1c, b1c):
    return pl.pallas_call(
        _node_update_body,
        grid=(_GRID,),
        in_specs=[_row_spec(9), _row_spec(1), _row_spec(1)]
        + _wspecs(W2a, b2a, W2b, b2b, W1a, b1a, W1b, b1b, W1c, b1c),
        out_specs=[_row_spec(9), _row_spec(1)],
        out_shape=[
            jax.ShapeDtypeStruct((_N, 9), jnp.float32),
            jax.ShapeDtypeStruct((_NPAD, 1), jnp.float32),
        ],
    )(x, p0, p1, W2a, b2a, W2b, b2b, W1a, b1a, W1b, b1b, W1c, b1c)


def _node_final(x, p0, p1, W2a, b2a, W2b, b2b, Wh1, bh1, Wh2, bh2):
    return pl.pallas_call(
        _node_final_body,
        grid=(_GRID,),
        in_specs=[_row_spec(9), _row_spec(1), _row_spec(1)]
        + _wspecs(W2a, b2a, W2b, b2b, Wh1, bh1, Wh2, bh2),
        out_specs=_row_spec(1),
        out_shape=jax.ShapeDtypeStruct((_N, 1), jnp.float32),
    )(x, p0, p1, W2a, b2a, W2b, b2b, Wh1, bh1, Wh2, bh2)


def kernel(x, edge_index, edge_attr, W1a, b1a, W1b, b1b, W1c, b1c,
           W2a, b2a, W2b, b2b, Wh1, bh1, Wh2, bh2):
    src = edge_index[0]
    dst = edge_index[1]
    attr = edge_attr.reshape(_E)

    b1a2 = b1a.reshape(1, -1)
    b1b2 = b1b.reshape(1, -1)
    b1c2 = b1c.reshape(1, -1)
    b2a2 = b2a.reshape(1, -1)
    b2b2 = b2b.reshape(1, -1)
    bh12 = bh1.reshape(1, -1)
    bh22 = bh2.reshape(1, -1)

    g = _node_gate(x, W1a, b1a2, W1b, b1b2, W1c, b1c2)

    xk = x
    for _ in range(2):
        p0, p1 = _edge_agg(g.reshape(_NPAD), src, dst, attr)
        p0 = p0.reshape(_NPAD, 1)
        p1 = p1.reshape(_NPAD, 1)
        xk, g = _node_update(xk, p0, p1, W2a, b2a2, W2b, b2b2,
                             W1a, b1a2, W1b, b1b2, W1c, b1c2)

    p0, p1 = _edge_agg(g.reshape(_NPAD), src, dst, attr)
    p0 = p0.reshape(_NPAD, 1)
    p1 = p1.reshape(_NPAD, 1)
    return _node_final(xk, p0, p1, W2a, b2a2, W2b, b2b2, Wh1, bh12, Wh2, bh22)


# speedup vs baseline: 180.1743x; 1.9767x over previous
import functools

import jax
import jax.numpy as jnp
from jax import lax
from jax.experimental import pallas as pl
from jax.experimental.pallas import tpu as pltpu
from jax.experimental.pallas import tpu_sc as plsc

_N = 100000
_E = 6400000
_GES = 8

_NW = 32
_EW = _E // _NW
_C = 8000
_NCHUNK = _EW // _C
_NPAD = 102400
_SLICE = _NPAD // 16

def _make_edge_agg():
    mesh = plsc.VectorSubcoreMesh(core_axis_name="c", subcore_axis_name="s")

    @functools.partial(
        pl.kernel,
        out_type=(
            jax.ShapeDtypeStruct((_NPAD,), jnp.float32),
            jax.ShapeDtypeStruct((_NPAD,), jnp.float32),
        ),
        mesh=mesh,
        scratch_types=[
            pltpu.VMEM((_C,), jnp.int32),
            pltpu.VMEM((_C,), jnp.int32),
            pltpu.VMEM((_C,), jnp.int32),
            pltpu.VMEM((_C,), jnp.int32),
            pltpu.VMEM((_C,), jnp.float32),
            pltpu.VMEM((_C,), jnp.float32),
            pltpu.VMEM((_C,), jnp.float32),
            pltpu.VMEM((_C,), jnp.float32),
            pltpu.VMEM_SHARED((_NPAD,), jnp.float32),
            pltpu.VMEM_SHARED((_NPAD,), jnp.float32),
            pltpu.SemaphoreType.DMA,
            pltpu.SemaphoreType.DMA,
            pltpu.SemaphoreType.DMA,
        ],
    )
    def edge_agg(g_hbm, src_hbm, dst_hbm, attr_hbm, out0_hbm, out1_hbm,
                 src0, src1, dst0, dst1, attr0, attr1, msg0, msg1,
                 acc_sh, g_sh, lin0, lin1, ga):
        c = lax.axis_index("c")
        s = lax.axis_index("s")
        wid = s * 2 + c
        ebase = wid * _EW
        SRC = [src0, src1]
        DST = [dst0, dst1]
        ATTR = [attr0, attr1]
        MSG = [msg0, msg1]
        LIN = [lin0, lin1]

        pltpu.sync_copy(g_hbm.at[pl.ds(s * _SLICE, _SLICE)],
                        msg0.at[pl.ds(0, _SLICE)])
        pltpu.sync_copy(msg0.at[pl.ds(0, _SLICE)],
                        g_sh.at[pl.ds(s * _SLICE, _SLICE)])

        @functools.partial(plsc.parallel_loop, 0, _SLICE // 16 + 1)
        def _zero(i):
            msg0[pl.ds(i * 16, 16)] = jnp.zeros((16,), jnp.float32)

        pltpu.sync_copy(msg0.at[pl.ds(0, _SLICE)],
                        acc_sh.at[pl.ds(s * _SLICE, _SLICE)])
        plsc.subcore_barrier()

        def _lin_start(k, b):
            base = pl.multiple_of(ebase + k * _C, 8)
            pltpu.async_copy(src_hbm.at[pl.ds(base, _C)], SRC[b], LIN[b])
            pltpu.async_copy(attr_hbm.at[pl.ds(base, _C)], ATTR[b], LIN[b])
            pltpu.async_copy(dst_hbm.at[pl.ds(base, _C)], DST[b], LIN[b])

        def _lin_wait(k, b):
            base = pl.multiple_of(ebase + k * _C, 8)
            pltpu.make_async_copy(src_hbm.at[pl.ds(base, _C)], SRC[b], LIN[b]).wait()
            pltpu.make_async_copy(attr_hbm.at[pl.ds(base, _C)], ATTR[b], LIN[b]).wait()
            pltpu.make_async_copy(dst_hbm.at[pl.ds(base, _C)], DST[b], LIN[b]).wait()

        def _substep(k, b, issue_next):
            _lin_wait(k, b)
            gather = pltpu.async_copy(g_sh.at[SRC[b]], MSG[b], ga)
            if issue_next:
                _lin_start(k + 1, 1 - b)
            gather.wait()

            @functools.partial(plsc.parallel_loop, 0, _C // 16, unroll=8)
            def _mul(i):
                MSG[b][pl.ds(i * 16, 16)] = (
                    MSG[b][pl.ds(i * 16, 16)] * ATTR[b][pl.ds(i * 16, 16)]
                )

            pltpu.sync_copy(MSG[b], acc_sh.at[DST[b]], add=True)

        _lin_start(0, 0)
        _substep(0, 0, issue_next=True)

        def _pair(j, carry):
            _substep(2 * j + 1, 1, issue_next=True)
            _substep(2 * j + 2, 0, issue_next=True)
            return carry

        lax.fori_loop(0, (_NCHUNK - 3) // 2, _pair, 0)

        _substep(_NCHUNK - 2, 1, issue_next=True)
        _substep(_NCHUNK - 1, 0, issue_next=False)
        plsc.subcore_barrier()

        def _drain(out_hbm):
            pltpu.sync_copy(acc_sh.at[pl.ds(s * _SLICE, _SLICE)],
                            msg0.at[pl.ds(0, _SLICE)])
            pltpu.sync_copy(msg0.at[pl.ds(0, _SLICE)],
                            out_hbm.at[pl.ds(s * _SLICE, _SLICE)])

        @pl.when(c == 0)
        def _():
            _drain(out0_hbm)

        @pl.when(c == 1)
        def _():
            _drain(out1_hbm)

    return edge_agg


_edge_agg_impl = None


def _edge_agg(g, src, dst, attr):
    global _edge_agg_impl
    if _edge_agg_impl is None:
        _edge_agg_impl = _make_edge_agg()
    return _edge_agg_impl(g, src, dst, attr)


_BL = 10240
_BR = _BL // 128
_NR = _NPAD // 128
_TGRID = _NPAD // _BL


def _mlp1_t(xt, W1aT, b1a, W1bT, b1b, W1cT, b1c):
    h = jnp.maximum(jnp.dot(W1aT, xt, preferred_element_type=jnp.float32) + b1a, 0.0)
    h = jnp.maximum(jnp.dot(W1bT, h, preferred_element_type=jnp.float32) + b1b, 0.0)
    z = jnp.dot(W1cT, h, preferred_element_type=jnp.float32) + b1c
    return 1.0 / (1.0 + jnp.exp(-z))


def _node_gate_body(x_ref, W1aT, b1a, W1bT, b1b, W1cT, b1c, g_ref):
    g = _mlp1_t(x_ref[...], W1aT[...], b1a[...], W1bT[...], b1b[...],
                W1cT[...], b1c[...])
    g_ref[...] = g.reshape(_BR, 128)


def _mlp2_t(x9, agg, W2aT, b2a, W2bT, b2b):
    tmp = jnp.concatenate([x9, agg], axis=0)
    c1 = jnp.maximum(jnp.dot(W2aT, tmp, preferred_element_type=jnp.float32) + b2a, 0.0)
    return jnp.maximum(jnp.dot(W2bT, c1, preferred_element_type=jnp.float32) + b2b, 0.0)


def _node_update_body(x_ref, p0_ref, p1_ref, W2aT, b2a, W2bT, b2b,
                      W1aT, b1a, W1bT, b1b, W1cT, b1c, xn_ref, g_ref):
    agg = (p0_ref[...] + p1_ref[...]).reshape(1, _BL)
    c2 = _mlp2_t(x_ref[0:9, :], agg, W2aT[...], b2a[...], W2bT[...], b2b[...])
    xn = jnp.concatenate(
        [x_ref[0:1, :], c2, jnp.zeros((7, c2.shape[1]), jnp.float32)], axis=0)
    xn_ref[...] = xn
    g = _mlp1_t(xn, W1aT[...], b1a[...], W1bT[...], b1b[...],
                W1cT[...], b1c[...])
    g_ref[...] = g.reshape(_BR, 128)


def _node_final_body(x_ref, p0_ref, p1_ref, W2aT, b2a, W2bT, b2b,
                     Wh1T, bh1, Wh2T, bh2, out_ref):
    agg = (p0_ref[...] + p1_ref[...]).reshape(1, _BL)
    c2 = _mlp2_t(x_ref[0:9, :], agg, W2aT[...], b2a[...], W2bT[...], b2b[...])
    h = jnp.maximum(jnp.dot(Wh1T[...], c2, preferred_element_type=jnp.float32) + bh1[...], 0.0)
    z = jnp.dot(Wh2T[...], h, preferred_element_type=jnp.float32) + bh2[...]
    out_ref[...] = (1.0 / (1.0 + jnp.exp(-z))).reshape(_BR, 128)


def _col_spec(rows):
    return pl.BlockSpec((rows, _BL), lambda i: (0, i))


def _pk_spec():
    return pl.BlockSpec((_BR, 128), lambda i: (i, 0))


def _full_spec(shape):
    ndim = len(shape)
    return pl.BlockSpec(shape, lambda i: (0,) * ndim)


def _wspecs(*ws):
    return [_full_spec(w.shape) for w in ws]


def _node_gate(xt, W1aT, b1a, W1bT, b1b, W1cT, b1c):
    return pl.pallas_call(
        _node_gate_body,
        grid=(_TGRID,),
        in_specs=[_col_spec(16)] + _wspecs(W1aT, b1a, W1bT, b1b, W1cT, b1c),
        out_specs=_pk_spec(),
        out_shape=jax.ShapeDtypeStruct((_NR, 128), jnp.float32),
    )(xt, W1aT, b1a, W1bT, b1b, W1cT, b1c)


def _node_update(xt, p0, p1, W2aT, b2a, W2bT, b2b, W1aT, b1a, W1bT, b1b, W1cT, b1c):
    return pl.pallas_call(
        _node_update_body,
        grid=(_TGRID,),
        in_specs=[_col_spec(16), _pk_spec(), _pk_spec()]
        + _wspecs(W2aT, b2a, W2bT, b2b, W1aT, b1a, W1bT, b1b, W1cT, b1c),
        out_specs=[_col_spec(16), _pk_spec()],
        out_shape=[
            jax.ShapeDtypeStruct((16, _NPAD), jnp.float32),
            jax.ShapeDtypeStruct((_NR, 128), jnp.float32),
        ],
    )(xt, p0, p1, W2aT, b2a, W2bT, b2b, W1aT, b1a, W1bT, b1b, W1cT, b1c)


def _node_final(xt, p0, p1, W2aT, b2a, W2bT, b2b, Wh1T, bh1, Wh2T, bh2):
    return pl.pallas_call(
        _node_final_body,
        grid=(_TGRID,),
        in_specs=[_col_spec(16), _pk_spec(), _pk_spec()]
        + _wspecs(W2aT, b2a, W2bT, b2b, Wh1T, bh1, Wh2T, bh2),
        out_specs=_pk_spec(),
        out_shape=jax.ShapeDtypeStruct((_NR, 128), jnp.float32),
    )(xt, p0, p1, W2aT, b2a, W2bT, b2b, Wh1T, bh1, Wh2T, bh2)


def kernel(x, edge_index, edge_attr, W1a, b1a, W1b, b1b, W1c, b1c,
           W2a, b2a, W2b, b2b, Wh1, bh1, Wh2, bh2):
    src = edge_index[0]
    dst = edge_index[1]
    attr = edge_attr.reshape(_E)

    xt = jnp.pad(x, ((0, _NPAD - _N), (0, 7))).T
    W1aT = jnp.pad(W1a, ((0, 7), (0, 0))).T
    W1bT = W1b.T
    W1cT = W1c.T
    W2aT = W2a.T
    W2bT = W2b.T
    Wh1T = Wh1.T
    Wh2T = Wh2.T
    b1ac = b1a.reshape(-1, 1)
    b1bc = b1b.reshape(-1, 1)
    b1cc = b1c.reshape(-1, 1)
    b2ac = b2a.reshape(-1, 1)
    b2bc = b2b.reshape(-1, 1)
    bh1c = bh1.reshape(-1, 1)
    bh2c = bh2.reshape(-1, 1)

    g = _node_gate(xt, W1aT, b1ac, W1bT, b1bc, W1cT, b1cc)

    for _ in range(2):
        p0, p1 = _edge_agg(g.reshape(_NPAD), src, dst, attr)
        xt, g = _node_update(xt, p0.reshape(_NR, 128), p1.reshape(_NR, 128),
                             W2aT, b2ac, W2bT, b2bc,
                             W1aT, b1ac, W1bT, b1bc, W1cT, b1cc)

    p0, p1 = _edge_agg(g.reshape(_NPAD), src, dst, attr)
    out_t = _node_final(xt, p0.reshape(_NR, 128), p1.reshape(_NR, 128),
                        W2aT, b2ac, W2bT, b2bc, Wh1T, bh1c, Wh2T, bh2c)
    return out_t.reshape(_NPAD)[:_N].reshape(_N, 1)
